# baseline (speedup 1.0000x reference)
import math

import jax
import jax.numpy as jnp
from jax.experimental import pallas as pl
from jax.experimental.pallas import tpu as pltpu

N = 8192
NFEAT = 128
NHID = 32
NLAYERS = 4
M = 16
LAMDA = 0.5
ALPHA = 0.1

BR_A = 256
NRB_A = N // BR_A
BR_B = 512
NRB_B = N // BR_B

_THETAS = [math.log(LAMDA / l + 1.0) for l in range(1, NLAYERS + 1)]


def _split_pair(val):
    v_hi = val.astype(jnp.bfloat16)
    v_lo = (val - v_hi.astype(jnp.float32)).astype(jnp.bfloat16)
    return v_hi, v_lo


def _body_a(x_ref, wf_ref, adj_ref, wadj_ref, convw_ref, fc0w_ref, fc0b_ref,
            h0_ref, cur1_ref, ca_ref, cw_ref,
            h0_s):
    l = pl.program_id(0)
    rb = pl.program_id(1)
    r0 = rb * BR_A

    @pl.when(l == 0)
    def _prologue():
        for b, ref in ((0, x_ref), (1, wf_ref)):
            h = jnp.maximum(ref[...] @ fc0w_ref[...] + fc0b_ref[...], 0.0)
            h0_s[b, pl.ds(r0, BR_A), :] = h
            h0_ref[b, :, :] = h

    @pl.when(l == 1)
    def _layer1():
        theta = jnp.float32(_THETAS[0])
        w = convw_ref[0]
        for b, aref, cref in ((0, adj_ref, ca_ref), (1, wadj_ref, cw_ref)):
            a = aref[...].astype(jnp.bfloat16)
            cref[...] = a
            h0_full = h0_s[b, :, :]
            h_hi, h_lo = _split_pair(h0_full)
            rhs = jnp.concatenate([h_hi, h_lo], axis=1)
            hh = jnp.dot(a, rhs, preferred_element_type=jnp.float32)
            hi = hh[:, :NHID] + hh[:, NHID:]
            h0_rows = h0_s[b, pl.ds(r0, BR_A), :]
            sup = (1.0 - ALPHA) * hi + ALPHA * h0_rows
            out = theta * jnp.dot(sup, w, preferred_element_type=jnp.float32) \
                + (1.0 - theta) * sup + h0_rows
            new = jnp.maximum(out, 0.0)
            n_hi, n_lo = _split_pair(new)
            cur1_ref[b, :, 0:NHID] = n_hi
            cur1_ref[b, :, NHID:2 * NHID] = n_lo


def _body_b(mut_ref,
            ca_ref, cw_ref, h0_ref, cur1_ref, convw_ref, fcwt_ref,
            fcb_ref, fc2w_ref, fc2b_ref, aux_ref,
            o1_ref, o2_ref,
            cur_s, fin_s):
    l = pl.program_id(0)
    rb = pl.program_id(1)
    r0 = rb * BR_B
    NL = NLAYERS - 1

    @pl.when((l == 0) & (rb == 0))
    def _seed():
        cur_s[0, :, :, :] = cur1_ref[...]

    rp = l % 2
    wp = (l + 1) % 2
    theta = jnp.where(l == 0, _THETAS[1],
            jnp.where(l == 1, _THETAS[2], _THETAS[3])).astype(jnp.float32)
    w = convw_ref[l + 1]
    for b, cref in ((0, ca_ref), (1, cw_ref)):
        a = cref[...]
        rhs = cur_s[rp, b, :, :]
        hh = jnp.dot(a, rhs, preferred_element_type=jnp.float32)
        hi = hh[:, :NHID] + hh[:, NHID:]
        sup = (1.0 - ALPHA) * hi + ALPHA * h0_ref[b, pl.ds(r0, BR_B), :]
        blk = cur_s[rp, b, pl.ds(r0, BR_B), :].astype(jnp.float32)
        inp_rows = blk[:, :NHID] + blk[:, NHID:]
        out = theta * jnp.dot(sup, w, preferred_element_type=jnp.float32) \
            + (1.0 - theta) * sup + inp_rows
        new = jnp.maximum(out, 0.0)
        n_hi, n_lo = _split_pair(new)
        cur_s[wp, b, pl.ds(r0, BR_B), 0:NHID] = n_hi
        cur_s[wp, b, pl.ds(r0, BR_B), NHID:2 * NHID] = n_lo

        @pl.when(l == NL - 1)
        def _():
            fin_s[b, pl.ds(r0, BR_B), :] = new

    @pl.when((l == NL - 1) & (rb == NRB_B - 1))
    def _head():
        sums = []
        for b in (0, 1):
            acc = jnp.zeros((1, NHID), dtype=jnp.float32)
            for i in range(M):
                acc = acc + fin_s[b, pl.ds(mut_ref[i], 1), :]
            sums.append(acc)
        differ = sums[0] - sums[1]
        lid = jnp.sum(differ * fcwt_ref[...], axis=1, keepdims=True) \
            + fcb_ref[0]
        o2_ref[...] = lid
        relu_lid = jnp.maximum(lid, 0.0)
        o1_ref[...] = relu_lid * fc2w_ref[0] \
            + (aux_ref[4] * M) * fc2w_ref[1] \
            + (aux_ref[5] * M) * fc2w_ref[2] + fc2b_ref[0]


def kernel(x, adj, wild_adj, wild_feature, nodes, mutaion_site, aux,
           fc0_w, fc0_b, conv_W, fc_w, fc_b, fc2_w, fc2_b):
    del nodes
    f32 = jnp.float32

    in_specs_a = [
        pl.BlockSpec((BR_A, NFEAT), lambda l, rb: (jnp.where(l == 0, rb, 0), 0)),
        pl.BlockSpec((BR_A, NFEAT), lambda l, rb: (jnp.where(l == 0, rb, 0), 0)),
        pl.BlockSpec((BR_A, N), lambda l, rb: (jnp.where(l == 0, 0, rb), 0)),
        pl.BlockSpec((BR_A, N), lambda l, rb: (jnp.where(l == 0, 0, rb), 0)),
        pl.BlockSpec((NLAYERS, NHID, NHID), lambda l, rb: (0, 0, 0)),
        pl.BlockSpec((NFEAT, NHID), lambda l, rb: (0, 0)),
        pl.BlockSpec((1, NHID), lambda l, rb: (0, 0)),
    ]
    out_specs_a = [
        pl.BlockSpec((2, BR_A, NHID),
                     lambda l, rb: (0, jnp.where(l == 0, rb, NRB_A - 1), 0)),
        pl.BlockSpec((2, BR_A, 2 * NHID), lambda l, rb: (0, jnp.where(l == 0, 0, rb), 0)),
        pl.BlockSpec((BR_A, N), lambda l, rb: (jnp.where(l == 0, 0, rb), 0)),
        pl.BlockSpec((BR_A, N), lambda l, rb: (jnp.where(l == 0, 0, rb), 0)),
    ]
    h0, cur1, cache_a, cache_w = pl.pallas_call(
        _body_a,
        grid=(2, NRB_A),
        in_specs=in_specs_a,
        out_specs=out_specs_a,
        out_shape=[
            jax.ShapeDtypeStruct((2, N, NHID), f32),
            jax.ShapeDtypeStruct((2, N, 2 * NHID), jnp.bfloat16),
            jax.ShapeDtypeStruct((N, N), jnp.bfloat16),
            jax.ShapeDtypeStruct((N, N), jnp.bfloat16),
        ],
        scratch_shapes=[pltpu.VMEM((2, N, NHID), f32)],
        compiler_params=pltpu.CompilerParams(
            dimension_semantics=("arbitrary", "arbitrary"),
            vmem_limit_bytes=62 * 1024 * 1024,
        ),
    )(x, wild_feature, adj, wild_adj, conv_W, fc0_w, fc0_b.reshape(1, NHID))

    in_specs_b = [
        pl.BlockSpec((BR_B, N), lambda l, rb, *_: (rb, 0)),
        pl.BlockSpec((BR_B, N), lambda l, rb, *_: (rb, 0)),
        pl.BlockSpec((2, N, NHID), lambda l, rb, *_: (0, 0, 0)),
        pl.BlockSpec((2, N, 2 * NHID), lambda l, rb, *_: (0, 0, 0)),
        pl.BlockSpec((NLAYERS, NHID, NHID), lambda l, rb, *_: (0, 0, 0)),
        pl.BlockSpec((1, NHID), lambda l, rb, *_: (0, 0)),
        pl.BlockSpec(memory_space=pltpu.SMEM),
        pl.BlockSpec(memory_space=pltpu.SMEM),
        pl.BlockSpec(memory_space=pltpu.SMEM),
        pl.BlockSpec(memory_space=pltpu.SMEM),
    ]
    out_specs_b = [
        pl.BlockSpec((1, 1), lambda l, rb, *_: (0, 0)),
        pl.BlockSpec((1, 1), lambda l, rb, *_: (0, 0)),
    ]
    grid_spec_b = pltpu.PrefetchScalarGridSpec(
        num_scalar_prefetch=1,
        grid=(NLAYERS - 1, NRB_B),
        in_specs=in_specs_b,
        out_specs=out_specs_b,
        scratch_shapes=[
            pltpu.VMEM((2, 2, N, 2 * NHID), jnp.bfloat16),
            pltpu.VMEM((2, N, NHID), f32),
        ],
    )
    o1, o2 = pl.pallas_call(
        _body_b,
        grid_spec=grid_spec_b,
        out_shape=[
            jax.ShapeDtypeStruct((1, 1), f32),
            jax.ShapeDtypeStruct((1, 1), f32),
        ],
        compiler_params=pltpu.CompilerParams(
            dimension_semantics=("arbitrary", "arbitrary"),
            vmem_limit_bytes=62 * 1024 * 1024,
        ),
    )(mutaion_site.astype(jnp.int32),
      cache_a, cache_w, h0, cur1, conv_W, fc_w.reshape(1, NHID),
      fc_b, fc2_w.reshape(3), fc2_b, aux)
    return (o1.reshape(1), o2.reshape(1))


# speedup vs baseline: 1.2331x; 1.2331x over previous
import math

import jax
import jax.numpy as jnp
from jax.experimental import pallas as pl
from jax.experimental.pallas import tpu as pltpu

N = 8192
NFEAT = 128
NHID = 32
NLAYERS = 4
M = 16
LAMDA = 0.5
ALPHA = 0.1

BR_A = 256
NRB_A = N // BR_A
BR_B = 256
NRB_B = N // BR_B

_THETAS = [math.log(LAMDA / l + 1.0) for l in range(1, NLAYERS + 1)]


def _split_pair(val):
    v_hi = val.astype(jnp.bfloat16)
    v_lo = (val - v_hi.astype(jnp.float32)).astype(jnp.bfloat16)
    return v_hi, v_lo


def _body_a(x_ref, wf_ref, adj_ref, wadj_ref, convw_ref, fc0w_ref, fc0b_ref,
            h0_ref, cur1_ref, ca_ref, cw_ref,
            h0_s):
    l = pl.program_id(0)
    rb = pl.program_id(1)
    r0 = rb * BR_A

    @pl.when(l == 0)
    def _prologue():
        for b, ref in ((0, x_ref), (1, wf_ref)):
            h = jnp.maximum(ref[...] @ fc0w_ref[...] + fc0b_ref[...], 0.0)
            h0_s[b, pl.ds(r0, BR_A), :] = h
            h0_ref[b, :, :] = h

    @pl.when(l == 1)
    def _layer1():
        theta = jnp.float32(_THETAS[0])
        w = convw_ref[0]
        for b, aref, cref in ((0, adj_ref, ca_ref), (1, wadj_ref, cw_ref)):
            a = aref[...].astype(jnp.bfloat16)
            cref[...] = a
            h0_full = h0_s[b, :, :]
            h_hi, h_lo = _split_pair(h0_full)
            rhs = jnp.concatenate([h_hi, h_lo], axis=1)
            hh = jnp.dot(a, rhs, preferred_element_type=jnp.float32)
            hi = hh[:, :NHID] + hh[:, NHID:]
            h0_rows = h0_s[b, pl.ds(r0, BR_A), :]
            sup = (1.0 - ALPHA) * hi + ALPHA * h0_rows
            out = theta * jnp.dot(sup, w, preferred_element_type=jnp.float32) \
                + (1.0 - theta) * sup + h0_rows
            new = jnp.maximum(out, 0.0)
            n_hi, n_lo = _split_pair(new)
            cur1_ref[b, :, 0:NHID] = n_hi
            cur1_ref[b, :, NHID:2 * NHID] = n_lo


def _body_b(mut_ref,
            ca_ref, cw_ref, h0_ref, cur1_ref, convw_ref, fcwt_ref,
            fcb_ref, fc2w_ref, fc2b_ref, aux_ref,
            o1_ref, o2_ref,
            cur_s, fin_s):
    l = pl.program_id(0)
    rb = pl.program_id(1)
    r0 = rb * BR_B
    NL = NLAYERS - 1

    @pl.when((l == 0) & (rb == 0))
    def _seed():
        cur_s[0, :, :, :] = cur1_ref[...]

    rp = l % 2
    wp = (l + 1) % 2
    theta = jnp.where(l == 0, _THETAS[1],
            jnp.where(l == 1, _THETAS[2], _THETAS[3])).astype(jnp.float32)
    w = convw_ref[l + 1]
    for b, cref in ((0, ca_ref), (1, cw_ref)):
        a = cref[...]
        rhs = cur_s[rp, b, :, :]
        hh = jnp.dot(a, rhs, preferred_element_type=jnp.float32)
        hi = hh[:, :NHID] + hh[:, NHID:]
        sup = (1.0 - ALPHA) * hi + ALPHA * h0_ref[b, pl.ds(r0, BR_B), :]
        blk = cur_s[rp, b, pl.ds(r0, BR_B), :].astype(jnp.float32)
        inp_rows = blk[:, :NHID] + blk[:, NHID:]
        out = theta * jnp.dot(sup, w, preferred_element_type=jnp.float32) \
            + (1.0 - theta) * sup + inp_rows
        new = jnp.maximum(out, 0.0)
        n_hi, n_lo = _split_pair(new)
        cur_s[wp, b, pl.ds(r0, BR_B), 0:NHID] = n_hi
        cur_s[wp, b, pl.ds(r0, BR_B), NHID:2 * NHID] = n_lo

        @pl.when(l == NL - 1)
        def _():
            fin_s[b, pl.ds(r0, BR_B), :] = new

    @pl.when((l == NL - 1) & (rb == NRB_B - 1))
    def _head():
        sums = []
        for b in (0, 1):
            acc = jnp.zeros((1, NHID), dtype=jnp.float32)
            for i in range(M):
                acc = acc + fin_s[b, pl.ds(mut_ref[i], 1), :]
            sums.append(acc)
        differ = sums[0] - sums[1]
        lid = jnp.sum(differ * fcwt_ref[...], axis=1, keepdims=True) \
            + fcb_ref[0]
        o2_ref[...] = lid
        relu_lid = jnp.maximum(lid, 0.0)
        o1_ref[...] = relu_lid * fc2w_ref[0] \
            + (aux_ref[4] * M) * fc2w_ref[1] \
            + (aux_ref[5] * M) * fc2w_ref[2] + fc2b_ref[0]


def kernel(x, adj, wild_adj, wild_feature, nodes, mutaion_site, aux,
           fc0_w, fc0_b, conv_W, fc_w, fc_b, fc2_w, fc2_b):
    del nodes
    f32 = jnp.float32

    in_specs_a = [
        pl.BlockSpec((BR_A, NFEAT), lambda l, rb: (jnp.where(l == 0, rb, 0), 0)),
        pl.BlockSpec((BR_A, NFEAT), lambda l, rb: (jnp.where(l == 0, rb, 0), 0)),
        pl.BlockSpec((BR_A, N), lambda l, rb: (jnp.where(l == 0, 0, rb), 0)),
        pl.BlockSpec((BR_A, N), lambda l, rb: (jnp.where(l == 0, 0, rb), 0)),
        pl.BlockSpec((NLAYERS, NHID, NHID), lambda l, rb: (0, 0, 0)),
        pl.BlockSpec((NFEAT, NHID), lambda l, rb: (0, 0)),
        pl.BlockSpec((1, NHID), lambda l, rb: (0, 0)),
    ]
    out_specs_a = [
        pl.BlockSpec((2, BR_A, NHID),
                     lambda l, rb: (0, jnp.where(l == 0, rb, NRB_A - 1), 0)),
        pl.BlockSpec((2, BR_A, 2 * NHID), lambda l, rb: (0, jnp.where(l == 0, 0, rb), 0)),
        pl.BlockSpec((BR_A, N), lambda l, rb: (jnp.where(l == 0, 0, rb), 0)),
        pl.BlockSpec((BR_A, N), lambda l, rb: (jnp.where(l == 0, 0, rb), 0)),
    ]
    h0, cur1, cache_a, cache_w = pl.pallas_call(
        _body_a,
        grid=(2, NRB_A),
        in_specs=in_specs_a,
        out_specs=out_specs_a,
        out_shape=[
            jax.ShapeDtypeStruct((2, N, NHID), f32),
            jax.ShapeDtypeStruct((2, N, 2 * NHID), jnp.bfloat16),
            jax.ShapeDtypeStruct((N, N), jnp.bfloat16),
            jax.ShapeDtypeStruct((N, N), jnp.bfloat16),
        ],
        scratch_shapes=[pltpu.VMEM((2, N, NHID), f32)],
        compiler_params=pltpu.CompilerParams(
            dimension_semantics=("arbitrary", "arbitrary"),
            vmem_limit_bytes=62 * 1024 * 1024,
        ),
    )(x, wild_feature, adj, wild_adj, conv_W, fc0_w, fc0_b.reshape(1, NHID))

    in_specs_b = [
        pl.BlockSpec((BR_B, N), lambda l, rb, *_: (rb, 0)),
        pl.BlockSpec((BR_B, N), lambda l, rb, *_: (rb, 0)),
        pl.BlockSpec((2, N, NHID), lambda l, rb, *_: (0, 0, 0)),
        pl.BlockSpec((2, N, 2 * NHID), lambda l, rb, *_: (0, 0, 0)),
        pl.BlockSpec((NLAYERS, NHID, NHID), lambda l, rb, *_: (0, 0, 0)),
        pl.BlockSpec((1, NHID), lambda l, rb, *_: (0, 0)),
        pl.BlockSpec(memory_space=pltpu.SMEM),
        pl.BlockSpec(memory_space=pltpu.SMEM),
        pl.BlockSpec(memory_space=pltpu.SMEM),
        pl.BlockSpec(memory_space=pltpu.SMEM),
    ]
    out_specs_b = [
        pl.BlockSpec((1, 1), lambda l, rb, *_: (0, 0)),
        pl.BlockSpec((1, 1), lambda l, rb, *_: (0, 0)),
    ]
    grid_spec_b = pltpu.PrefetchScalarGridSpec(
        num_scalar_prefetch=1,
        grid=(NLAYERS - 1, NRB_B),
        in_specs=in_specs_b,
        out_specs=out_specs_b,
        scratch_shapes=[
            pltpu.VMEM((2, 2, N, 2 * NHID), jnp.bfloat16),
            pltpu.VMEM((2, N, NHID), f32),
        ],
    )
    o1, o2 = pl.pallas_call(
        _body_b,
        grid_spec=grid_spec_b,
        out_shape=[
            jax.ShapeDtypeStruct((1, 1), f32),
            jax.ShapeDtypeStruct((1, 1), f32),
        ],
        compiler_params=pltpu.CompilerParams(
            dimension_semantics=("arbitrary", "arbitrary"),
            vmem_limit_bytes=62 * 1024 * 1024,
        ),
    )(mutaion_site.astype(jnp.int32),
      cache_a, cache_w, h0, cur1, conv_W, fc_w.reshape(1, NHID),
      fc_b, fc2_w.reshape(3), fc2_b, aux)
    return (o1.reshape(1), o2.reshape(1))


# speedup vs baseline: 1.2657x; 1.0264x over previous
import math

import jax
import jax.numpy as jnp
from jax.experimental import pallas as pl
from jax.experimental.pallas import tpu as pltpu

N = 8192
NFEAT = 128
NHID = 32
NLAYERS = 4
M = 16
LAMDA = 0.5
ALPHA = 0.1

BR_A = 256
NRB_A = N // BR_A
BR_B = 256
NRB_B = N // BR_B

_THETAS = [math.log(LAMDA / l + 1.0) for l in range(1, NLAYERS + 1)]


def _split_pair(val):
    v_hi = val.astype(jnp.bfloat16)
    v_lo = (val - v_hi.astype(jnp.float32)).astype(jnp.bfloat16)
    return v_hi, v_lo


def _body_a(x_ref, wf_ref, adj_ref, wadj_ref, convw_ref, fc0w_ref, fc0b_ref,
            h0_ref, cur1_ref, ca_ref, cw_ref,
            h0_s, rhs_s):
    l = pl.program_id(0)
    rb = pl.program_id(1)
    r0 = rb * BR_A

    @pl.when(l == 0)
    def _prologue():
        for b, ref in ((0, x_ref), (1, wf_ref)):
            h = jnp.maximum(ref[...] @ fc0w_ref[...] + fc0b_ref[...], 0.0)
            h0_s[b, pl.ds(r0, BR_A), :] = h
            h0_ref[b, :, :] = h
            h_hi, h_lo = _split_pair(h)
            rhs_s[b, pl.ds(r0, BR_A), 0:NHID] = h_hi
            rhs_s[b, pl.ds(r0, BR_A), NHID:2 * NHID] = h_lo

    @pl.when(l == 1)
    def _layer1():
        theta = jnp.float32(_THETAS[0])
        w = convw_ref[0]
        a0 = adj_ref[...].astype(jnp.bfloat16)
        a1 = wadj_ref[...].astype(jnp.bfloat16)
        ca_ref[...] = a0
        cw_ref[...] = a1
        hh0 = jnp.dot(a0, rhs_s[0, :, :], preferred_element_type=jnp.float32)
        hh1 = jnp.dot(a1, rhs_s[1, :, :], preferred_element_type=jnp.float32)
        for b, hh in ((0, hh0), (1, hh1)):
            hi = hh[:, :NHID] + hh[:, NHID:]
            h0_rows = h0_s[b, pl.ds(r0, BR_A), :]
            sup = (1.0 - ALPHA) * hi + ALPHA * h0_rows
            out = theta * jnp.dot(sup, w, preferred_element_type=jnp.float32) \
                + (1.0 - theta) * sup + h0_rows
            new = jnp.maximum(out, 0.0)
            n_hi, n_lo = _split_pair(new)
            cur1_ref[b, :, 0:NHID] = n_hi
            cur1_ref[b, :, NHID:2 * NHID] = n_lo


def _body_b(mut_ref,
            ca_ref, cw_ref, h0_ref, cur1_ref, convw_ref, fcwt_ref,
            fcb_ref, fc2w_ref, fc2b_ref, aux_ref,
            o1_ref, o2_ref,
            cur_s, fin_s):
    l = pl.program_id(0)
    rb = pl.program_id(1)
    r0 = rb * BR_B
    NL = NLAYERS - 1

    @pl.when((l == 0) & (rb == 0))
    def _seed():
        cur_s[0, :, :, :] = cur1_ref[...]

    rp = l % 2
    wp = (l + 1) % 2
    theta = jnp.where(l == 0, _THETAS[1],
            jnp.where(l == 1, _THETAS[2], _THETAS[3])).astype(jnp.float32)
    w = convw_ref[l + 1]
    hh0 = jnp.dot(ca_ref[...], cur_s[rp, 0, :, :],
                  preferred_element_type=jnp.float32)
    hh1 = jnp.dot(cw_ref[...], cur_s[rp, 1, :, :],
                  preferred_element_type=jnp.float32)
    for b, hh in ((0, hh0), (1, hh1)):
        hi = hh[:, :NHID] + hh[:, NHID:]
        sup = (1.0 - ALPHA) * hi + ALPHA * h0_ref[b, pl.ds(r0, BR_B), :]
        blk = cur_s[rp, b, pl.ds(r0, BR_B), :].astype(jnp.float32)
        inp_rows = blk[:, :NHID] + blk[:, NHID:]
        out = theta * jnp.dot(sup, w, preferred_element_type=jnp.float32) \
            + (1.0 - theta) * sup + inp_rows
        new = jnp.maximum(out, 0.0)
        n_hi, n_lo = _split_pair(new)
        cur_s[wp, b, pl.ds(r0, BR_B), 0:NHID] = n_hi
        cur_s[wp, b, pl.ds(r0, BR_B), NHID:2 * NHID] = n_lo

        @pl.when(l == NL - 1)
        def _():
            fin_s[b, pl.ds(r0, BR_B), :] = new

    @pl.when((l == NL - 1) & (rb == NRB_B - 1))
    def _head():
        sums = []
        for b in (0, 1):
            acc = jnp.zeros((1, NHID), dtype=jnp.float32)
            for i in range(M):
                acc = acc + fin_s[b, pl.ds(mut_ref[i], 1), :]
            sums.append(acc)
        differ = sums[0] - sums[1]
        lid = jnp.sum(differ * fcwt_ref[...], axis=1, keepdims=True) \
            + fcb_ref[0]
        o2_ref[...] = lid
        relu_lid = jnp.maximum(lid, 0.0)
        o1_ref[...] = relu_lid * fc2w_ref[0] \
            + (aux_ref[4] * M) * fc2w_ref[1] \
            + (aux_ref[5] * M) * fc2w_ref[2] + fc2b_ref[0]


def kernel(x, adj, wild_adj, wild_feature, nodes, mutaion_site, aux,
           fc0_w, fc0_b, conv_W, fc_w, fc_b, fc2_w, fc2_b):
    del nodes
    f32 = jnp.float32

    in_specs_a = [
        pl.BlockSpec((BR_A, NFEAT), lambda l, rb: (jnp.where(l == 0, rb, 0), 0)),
        pl.BlockSpec((BR_A, NFEAT), lambda l, rb: (jnp.where(l == 0, rb, 0), 0)),
        pl.BlockSpec((BR_A, N), lambda l, rb: (jnp.where(l == 0, 0, rb), 0)),
        pl.BlockSpec((BR_A, N), lambda l, rb: (jnp.where(l == 0, 0, rb), 0)),
        pl.BlockSpec((NLAYERS, NHID, NHID), lambda l, rb: (0, 0, 0)),
        pl.BlockSpec((NFEAT, NHID), lambda l, rb: (0, 0)),
        pl.BlockSpec((1, NHID), lambda l, rb: (0, 0)),
    ]
    out_specs_a = [
        pl.BlockSpec((2, BR_A, NHID),
                     lambda l, rb: (0, jnp.where(l == 0, rb, NRB_A - 1), 0)),
        pl.BlockSpec((2, BR_A, 2 * NHID), lambda l, rb: (0, jnp.where(l == 0, 0, rb), 0)),
        pl.BlockSpec((BR_A, N), lambda l, rb: (jnp.where(l == 0, 0, rb), 0)),
        pl.BlockSpec((BR_A, N), lambda l, rb: (jnp.where(l == 0, 0, rb), 0)),
    ]
    h0, cur1, cache_a, cache_w = pl.pallas_call(
        _body_a,
        grid=(2, NRB_A),
        in_specs=in_specs_a,
        out_specs=out_specs_a,
        out_shape=[
            jax.ShapeDtypeStruct((2, N, NHID), f32),
            jax.ShapeDtypeStruct((2, N, 2 * NHID), jnp.bfloat16),
            jax.ShapeDtypeStruct((N, N), jnp.bfloat16),
            jax.ShapeDtypeStruct((N, N), jnp.bfloat16),
        ],
        scratch_shapes=[pltpu.VMEM((2, N, NHID), f32),
                        pltpu.VMEM((2, N, 2 * NHID), jnp.bfloat16)],
        compiler_params=pltpu.CompilerParams(
            dimension_semantics=("arbitrary", "arbitrary"),
            vmem_limit_bytes=62 * 1024 * 1024,
        ),
    )(x, wild_feature, adj, wild_adj, conv_W, fc0_w, fc0_b.reshape(1, NHID))

    in_specs_b = [
        pl.BlockSpec((BR_B, N), lambda l, rb, *_: (rb, 0)),
        pl.BlockSpec((BR_B, N), lambda l, rb, *_: (rb, 0)),
        pl.BlockSpec((2, N, NHID), lambda l, rb, *_: (0, 0, 0)),
        pl.BlockSpec((2, N, 2 * NHID), lambda l, rb, *_: (0, 0, 0)),
        pl.BlockSpec((NLAYERS, NHID, NHID), lambda l, rb, *_: (0, 0, 0)),
        pl.BlockSpec((1, NHID), lambda l, rb, *_: (0, 0)),
        pl.BlockSpec(memory_space=pltpu.SMEM),
        pl.BlockSpec(memory_space=pltpu.SMEM),
        pl.BlockSpec(memory_space=pltpu.SMEM),
        pl.BlockSpec(memory_space=pltpu.SMEM),
    ]
    out_specs_b = [
        pl.BlockSpec((1, 1), lambda l, rb, *_: (0, 0)),
        pl.BlockSpec((1, 1), lambda l, rb, *_: (0, 0)),
    ]
    grid_spec_b = pltpu.PrefetchScalarGridSpec(
        num_scalar_prefetch=1,
        grid=(NLAYERS - 1, NRB_B),
        in_specs=in_specs_b,
        out_specs=out_specs_b,
        scratch_shapes=[
            pltpu.VMEM((2, 2, N, 2 * NHID), jnp.bfloat16),
            pltpu.VMEM((2, N, NHID), f32),
        ],
    )
    o1, o2 = pl.pallas_call(
        _body_b,
        grid_spec=grid_spec_b,
        out_shape=[
            jax.ShapeDtypeStruct((1, 1), f32),
            jax.ShapeDtypeStruct((1, 1), f32),
        ],
        compiler_params=pltpu.CompilerParams(
            dimension_semantics=("arbitrary", "arbitrary"),
            vmem_limit_bytes=62 * 1024 * 1024,
        ),
    )(mutaion_site.astype(jnp.int32),
      cache_a, cache_w, h0, cur1, conv_W, fc_w.reshape(1, NHID),
      fc_b, fc2_w.reshape(3), fc2_b, aux)
    return (o1.reshape(1), o2.reshape(1))


# speedup vs baseline: 1.2957x; 1.0237x over previous
import math

import jax
import jax.numpy as jnp
from jax.experimental import pallas as pl
from jax.experimental.pallas import tpu as pltpu

N = 8192
NFEAT = 128
NHID = 32
NLAYERS = 4
M = 16
LAMDA = 0.5
ALPHA = 0.1

BR_A = 256
NRB_A = N // BR_A
PBR_A = 1024
NPB_A = N // PBR_A
BR_B = 256
NRB_B = N // BR_B

_THETAS = [math.log(LAMDA / l + 1.0) for l in range(1, NLAYERS + 1)]


def _split_pair(val):
    v_hi = val.astype(jnp.bfloat16)
    v_lo = (val - v_hi.astype(jnp.float32)).astype(jnp.bfloat16)
    return v_hi, v_lo


def _body_a(x_ref, wf_ref, adj_ref, wadj_ref, convw_ref, fc0w_ref, fc0b_ref,
            h0_ref, cur1_ref, ca_ref, cw_ref,
            rhs_s):
    s = pl.program_id(0)
    rb = s - NPB_A
    r0 = rb * BR_A

    @pl.when(s < NPB_A)
    def _prologue():
        r0p = s * PBR_A
        for b, ref in ((0, x_ref), (1, wf_ref)):
            h = jnp.maximum(ref[...] @ fc0w_ref[...] + fc0b_ref[...], 0.0)
            h_hi, h_lo = _split_pair(h)
            rhs_s[b, pl.ds(r0p, PBR_A), 0:NHID] = h_hi
            rhs_s[b, pl.ds(r0p, PBR_A), NHID:2 * NHID] = h_lo
            h0_ref[b, :, 0:NHID] = h_hi
            h0_ref[b, :, NHID:2 * NHID] = h_lo

    @pl.when(s >= NPB_A)
    def _layer1():
        theta = jnp.float32(_THETAS[0])
        w = convw_ref[0]
        a0 = adj_ref[...].astype(jnp.bfloat16)
        a1 = wadj_ref[...].astype(jnp.bfloat16)
        ca_ref[...] = a0
        cw_ref[...] = a1
        hh0 = jnp.dot(a0, rhs_s[0, :, :], preferred_element_type=jnp.float32)
        hh1 = jnp.dot(a1, rhs_s[1, :, :], preferred_element_type=jnp.float32)
        for b, hh in ((0, hh0), (1, hh1)):
            hi = hh[:, :NHID] + hh[:, NHID:]
            hp = rhs_s[b, pl.ds(r0, BR_A), :].astype(jnp.float32)
            h0_rows = hp[:, :NHID] + hp[:, NHID:]
            sup = (1.0 - ALPHA) * hi + ALPHA * h0_rows
            out = theta * jnp.dot(sup, w, preferred_element_type=jnp.float32) \
                + (1.0 - theta) * sup + h0_rows
            new = jnp.maximum(out, 0.0)
            n_hi, n_lo = _split_pair(new)
            cur1_ref[b, :, 0:NHID] = n_hi
            cur1_ref[b, :, NHID:2 * NHID] = n_lo


def _body_b(mut_ref,
            ca_ref, cw_ref, h0_ref, cur1_ref, convw_ref, fcwt_ref,
            fcb_ref, fc2w_ref, fc2b_ref, aux_ref,
            o1_ref, o2_ref,
            cur_s, fin_s):
    l = pl.program_id(0)
    rb = pl.program_id(1)
    r0 = rb * BR_B
    NL = NLAYERS - 1

    @pl.when((l == 0) & (rb == 0))
    def _seed():
        cur_s[0, :, :, :] = cur1_ref[...]

    rp = l % 2
    wp = (l + 1) % 2
    theta = jnp.where(l == 0, _THETAS[1],
            jnp.where(l == 1, _THETAS[2], _THETAS[3])).astype(jnp.float32)
    w = convw_ref[l + 1]
    hh0 = jnp.dot(ca_ref[...], cur_s[rp, 0, :, :],
                  preferred_element_type=jnp.float32)
    hh1 = jnp.dot(cw_ref[...], cur_s[rp, 1, :, :],
                  preferred_element_type=jnp.float32)
    for b, hh in ((0, hh0), (1, hh1)):
        hi = hh[:, :NHID] + hh[:, NHID:]
        h0p = h0_ref[b, pl.ds(r0, BR_B), :].astype(jnp.float32)
        h0_rows = h0p[:, :NHID] + h0p[:, NHID:]
        sup = (1.0 - ALPHA) * hi + ALPHA * h0_rows
        blk = cur_s[rp, b, pl.ds(r0, BR_B), :].astype(jnp.float32)
        inp_rows = blk[:, :NHID] + blk[:, NHID:]
        out = theta * jnp.dot(sup, w, preferred_element_type=jnp.float32) \
            + (1.0 - theta) * sup + inp_rows
        new = jnp.maximum(out, 0.0)
        n_hi, n_lo = _split_pair(new)
        cur_s[wp, b, pl.ds(r0, BR_B), 0:NHID] = n_hi
        cur_s[wp, b, pl.ds(r0, BR_B), NHID:2 * NHID] = n_lo

        @pl.when(l == NL - 1)
        def _():
            fin_s[b, pl.ds(r0, BR_B), :] = new

    @pl.when((l == NL - 1) & (rb == NRB_B - 1))
    def _head():
        sums = []
        for b in (0, 1):
            acc = jnp.zeros((1, NHID), dtype=jnp.float32)
            for i in range(M):
                acc = acc + fin_s[b, pl.ds(mut_ref[i], 1), :]
            sums.append(acc)
        differ = sums[0] - sums[1]
        lid = jnp.sum(differ * fcwt_ref[...], axis=1, keepdims=True) \
            + fcb_ref[0]
        o2_ref[...] = lid
        relu_lid = jnp.maximum(lid, 0.0)
        o1_ref[...] = relu_lid * fc2w_ref[0] \
            + (aux_ref[4] * M) * fc2w_ref[1] \
            + (aux_ref[5] * M) * fc2w_ref[2] + fc2b_ref[0]


def kernel(x, adj, wild_adj, wild_feature, nodes, mutaion_site, aux,
           fc0_w, fc0_b, conv_W, fc_w, fc_b, fc2_w, fc2_b):
    del nodes
    f32 = jnp.float32

    in_specs_a = [
        pl.BlockSpec((PBR_A, NFEAT), lambda s: (jnp.where(s < NPB_A, s, 0), 0)),
        pl.BlockSpec((PBR_A, NFEAT), lambda s: (jnp.where(s < NPB_A, s, 0), 0)),
        pl.BlockSpec((BR_A, N), lambda s: (jnp.maximum(s - NPB_A, 0), 0)),
        pl.BlockSpec((BR_A, N), lambda s: (jnp.maximum(s - NPB_A, 0), 0)),
        pl.BlockSpec((NLAYERS, NHID, NHID), lambda s: (0, 0, 0)),
        pl.BlockSpec((NFEAT, NHID), lambda s: (0, 0)),
        pl.BlockSpec((1, NHID), lambda s: (0, 0)),
    ]
    out_specs_a = [
        pl.BlockSpec((2, PBR_A, 2 * NHID),
                     lambda s: (0, jnp.where(s < NPB_A, s, NPB_A - 1), 0)),
        pl.BlockSpec((2, BR_A, 2 * NHID),
                     lambda s: (0, jnp.maximum(s - NPB_A, 0), 0)),
        pl.BlockSpec((BR_A, N), lambda s: (jnp.maximum(s - NPB_A, 0), 0)),
        pl.BlockSpec((BR_A, N), lambda s: (jnp.maximum(s - NPB_A, 0), 0)),
    ]
    h0, cur1, cache_a, cache_w = pl.pallas_call(
        _body_a,
        grid=(NPB_A + NRB_A,),
        in_specs=in_specs_a,
        out_specs=out_specs_a,
        out_shape=[
            jax.ShapeDtypeStruct((2, N, 2 * NHID), jnp.bfloat16),
            jax.ShapeDtypeStruct((2, N, 2 * NHID), jnp.bfloat16),
            jax.ShapeDtypeStruct((N, N), jnp.bfloat16),
            jax.ShapeDtypeStruct((N, N), jnp.bfloat16),
        ],
        scratch_shapes=[pltpu.VMEM((2, N, 2 * NHID), jnp.bfloat16)],
        compiler_params=pltpu.CompilerParams(
            dimension_semantics=("arbitrary",),
            vmem_limit_bytes=62 * 1024 * 1024,
        ),
    )(x, wild_feature, adj, wild_adj, conv_W, fc0_w, fc0_b.reshape(1, NHID))

    in_specs_b = [
        pl.BlockSpec((BR_B, N), lambda l, rb, *_: (rb, 0)),
        pl.BlockSpec((BR_B, N), lambda l, rb, *_: (rb, 0)),
        pl.BlockSpec((2, N, 2 * NHID), lambda l, rb, *_: (0, 0, 0)),
        pl.BlockSpec((2, N, 2 * NHID), lambda l, rb, *_: (0, 0, 0)),
        pl.BlockSpec((NLAYERS, NHID, NHID), lambda l, rb, *_: (0, 0, 0)),
        pl.BlockSpec((1, NHID), lambda l, rb, *_: (0, 0)),
        pl.BlockSpec(memory_space=pltpu.SMEM),
        pl.BlockSpec(memory_space=pltpu.SMEM),
        pl.BlockSpec(memory_space=pltpu.SMEM),
        pl.BlockSpec(memory_space=pltpu.SMEM),
    ]
    out_specs_b = [
        pl.BlockSpec((1, 1), lambda l, rb, *_: (0, 0)),
        pl.BlockSpec((1, 1), lambda l, rb, *_: (0, 0)),
    ]
    grid_spec_b = pltpu.PrefetchScalarGridSpec(
        num_scalar_prefetch=1,
        grid=(NLAYERS - 1, NRB_B),
        in_specs=in_specs_b,
        out_specs=out_specs_b,
        scratch_shapes=[
            pltpu.VMEM((2, 2, N, 2 * NHID), jnp.bfloat16),
            pltpu.VMEM((2, N, NHID), f32),
        ],
    )
    o1, o2 = pl.pallas_call(
        _body_b,
        grid_spec=grid_spec_b,
        out_shape=[
            jax.ShapeDtypeStruct((1, 1), f32),
            jax.ShapeDtypeStruct((1, 1), f32),
        ],
        compiler_params=pltpu.CompilerParams(
            dimension_semantics=("arbitrary", "arbitrary"),
            vmem_limit_bytes=96 * 1024 * 1024,
        ),
    )(mutaion_site.astype(jnp.int32),
      cache_a, cache_w, h0, cur1, conv_W, fc_w.reshape(1, NHID),
      fc_b, fc2_w.reshape(3), fc2_b, aux)
    return (o1.reshape(1), o2.reshape(1))
